# baseline (speedup 1.0000x reference)
import functools

import jax
import jax.numpy as jnp
from jax import lax
from jax.experimental import pallas as pl
from jax.experimental.pallas import tpu as pltpu
from jax.experimental.pallas import tpu_sc as plsc

_BLOCK_COLS = 49152


def _sc_lookup_body(ct_hbm, table_hbm, patt_hbm, ct_v, ttab_v, patt_v):
    c = lax.axis_index("c")
    s = lax.axis_index("s")

    @pl.when(jnp.logical_and(c == 0, s == 0))
    def _():
        pltpu.sync_copy(ct_hbm, ct_v)
        pltpu.sync_copy(table_hbm, ttab_v)
        ct16 = plsc.load_gather(ct_v, [jnp.zeros((16,), jnp.int32)])
        for j in range(64):
            v = plsc.load_gather(ttab_v, [ct16, jnp.full((16,), j, jnp.int32)])
            for l in range(8):
                patt_v[j, pl.ds(16 * l, 16)] = v
        pltpu.sync_copy(patt_v, patt_hbm)


def _tc_body(patt_ref, x_ref, o_ref):
    o_ref[...] = x_ref[...] + patt_ref[:, 0:1]


def kernel(x, cell_type_id, table):
    n, d = x.shape
    xt = x.T
    ct = cell_type_id.astype(jnp.int32)

    mesh = plsc.VectorSubcoreMesh(core_axis_name="c", subcore_axis_name="s")
    sc_lookup = functools.partial(
        pl.kernel,
        out_type=jax.ShapeDtypeStruct((d, 128), jnp.float32),
        mesh=mesh,
        scratch_types=[
            pltpu.VMEM((1,), jnp.int32),
            pltpu.VMEM(table.shape, jnp.float32),
            pltpu.VMEM((d, 128), jnp.float32),
        ],
        compiler_params=pltpu.CompilerParams(needs_layout_passes=False),
    )(_sc_lookup_body)
    patt = sc_lookup(ct, table)

    grid = pl.cdiv(n, _BLOCK_COLS)
    outt = pl.pallas_call(
        _tc_body,
        grid=(grid,),
        in_specs=[
            pl.BlockSpec((d, 128), lambda i: (0, 0)),
            pl.BlockSpec((d, _BLOCK_COLS), lambda i: (0, i)),
        ],
        out_specs=pl.BlockSpec((d, _BLOCK_COLS), lambda i: (0, i)),
        out_shape=jax.ShapeDtypeStruct((d, n), jnp.float32),
        compiler_params=pltpu.CompilerParams(
            dimension_semantics=("parallel",),
        ),
    )(patt, xt)
    return outt.T
